# baseline (speedup 1.0000x reference)
import functools

import jax
import jax.numpy as jnp
from jax import lax
from jax.experimental import pallas as pl
from jax.experimental.pallas import tpu as pltpu
from jax.experimental.pallas import tpu_sc as plsc

NUM_EMB_ROWS = 1000001
EMB_D = 64
BATCH = 16384
SEQ = 50

NC = 2
NS = 16
NW = NC * NS

BLK = 128
NBLOCKS = SEQ * (BATCH // BLK)
BL_PER_W = NBLOCKS // NW
NTI = BATCH // BLK
NBUF = 4
AHEAD = 2


@functools.partial(
    pl.kernel,
    out_type=jax.ShapeDtypeStruct((SEQ * 8, NTI, 8 * BLK), jnp.float32),
    mesh=plsc.VectorSubcoreMesh(core_axis_name="c", subcore_axis_name="s"),
    scratch_types=[
        pltpu.VMEM((BL_PER_W * BLK,), jnp.int32),
        pltpu.VMEM((NBUF, BLK, EMB_D), jnp.float32),
        pltpu.VMEM((NBUF, 8, 1, 8 * BLK), jnp.float32),
        [pltpu.SemaphoreType.DMA] * NBUF,
        [pltpu.SemaphoreType.DMA] * NBUF,
    ],
    compiler_params=pltpu.CompilerParams(
        use_tc_tiling_on_sc=False, needs_layout_passes=False
    ),
)
def _emb_lookup(x_hbm, table_hbm, out_hbm, idx_all, rows, planes, sg, so):
    wid = lax.axis_index("s") * NC + lax.axis_index("c")
    bl0 = wid * BL_PER_W
    pltpu.sync_copy(x_hbm.at[pl.ds(bl0 * BLK, BL_PER_W * BLK)], idx_all)

    lanes = lax.iota(jnp.int32, 16)

    def fire_g(t, b):
        pltpu.async_copy(
            table_hbm.at[idx_all.at[pl.ds(t * BLK, BLK)]], rows.at[b], sg[b]
        )

    def drain_g(b):
        pltpu.make_async_copy(
            table_hbm.at[idx_all.at[pl.ds(0, BLK)]], rows.at[b], sg[b]
        ).wait()

    def transpose(b):
        def tk_body(c2, carry):
            tk = c2 // 2
            kp0 = (c2 - tk * 2) * 4
            for kpo in range(4):
                kp = kp0 + kpo
                col = jnp.full((16,), kpo, jnp.int32) + (tk * 8 + kp0)
                for m in range(8):
                    v = plsc.load_gather(rows.at[b], [lanes + 16 * m, col])
                    planes[b, tk, 0, pl.ds(kp * BLK + 16 * m, 16)] = v
            return carry

        lax.fori_loop(0, 16, tk_body, 0)

    def fire_w(t, b):
        bl = bl0 + t
        j = bl // NTI
        ti = bl - j * NTI
        pltpu.async_copy(
            planes.at[b], out_hbm.at[pl.ds(j * 8, 8), pl.ds(ti, 1), :], so[b]
        )

    def drain_w(b):
        pltpu.make_async_copy(
            planes.at[b], out_hbm.at[pl.ds(0, 8), pl.ds(0, 1), :], so[b]
        ).wait()

    NPRO = NBUF + AHEAD
    for t in range(AHEAD):
        fire_g(t, t % NBUF)
    for t in range(NPRO):
        b = t % NBUF
        drain_g(b)
        fire_g(t + AHEAD, (t + AHEAD) % NBUF)
        if t >= NBUF:
            drain_w(b)
        transpose(b)
        fire_w(t, b)

    def loop_body(T, carry):
        t0 = NPRO + T * NBUF
        for i in range(NBUF):
            t = t0 + i
            b = (NPRO + i) % NBUF
            drain_g(b)
            fire_g(t + AHEAD, (NPRO + i + AHEAD) % NBUF)
            drain_w(b)
            transpose(b)
            fire_w(t, b)
        return carry

    lax.fori_loop(0, (BL_PER_W - NPRO - AHEAD) // NBUF, loop_body, 0)

    for t in range(BL_PER_W - AHEAD, BL_PER_W):
        b = t % NBUF
        drain_g(b)
        drain_w(b)
        transpose(b)
        fire_w(t, b)
    for t in range(BL_PER_W - NBUF, BL_PER_W):
        drain_w(t % NBUF)


def kernel(x, table):
    x_lin = x.astype(jnp.int32).T.reshape(-1)
    o5 = _emb_lookup(x_lin, table).reshape(SEQ, 8, NTI, 8, BLK)
    return o5.transpose((2, 4, 0, 1, 3)).reshape(BATCH, SEQ, EMB_D)


# speedup vs baseline: 1.7766x; 1.7766x over previous
import functools

import jax
import jax.numpy as jnp
from jax import lax
from jax.experimental import pallas as pl
from jax.experimental.pallas import tpu as pltpu
from jax.experimental.pallas import tpu_sc as plsc

NUM_EMB_ROWS = 1000001
EMB_D = 64
BATCH = 16384
SEQ = 50

NC = 2
NS = 16
NW = NC * NS

BLK = 128
NBLOCKS = SEQ * (BATCH // BLK)
BL_PER_W = NBLOCKS // NW
NTI = BATCH // BLK
NBUF = 4
AHEAD = 2


@functools.partial(
    pl.kernel,
    out_type=jax.ShapeDtypeStruct((SEQ * 8, NTI, 8 * BLK), jnp.float32),
    mesh=plsc.VectorSubcoreMesh(core_axis_name="c", subcore_axis_name="s"),
    scratch_types=[
        pltpu.VMEM((BL_PER_W * BLK,), jnp.int32),
        pltpu.VMEM((NBUF, BLK, EMB_D), jnp.float32),
        pltpu.VMEM((NBUF, 8, 1, 8 * BLK), jnp.float32),
        [pltpu.SemaphoreType.DMA] * NBUF,
        [pltpu.SemaphoreType.DMA] * NBUF,
    ],
    compiler_params=pltpu.CompilerParams(
        use_tc_tiling_on_sc=False, needs_layout_passes=False
    ),
)
def _emb_lookup(x_hbm, table_hbm, out_hbm, idx_all, rows, planes, sg, so):
    wid = lax.axis_index("s") * NC + lax.axis_index("c")
    bl0 = wid * BL_PER_W
    pltpu.sync_copy(x_hbm.at[pl.ds(bl0 * BLK, BL_PER_W * BLK)], idx_all)

    lanes = lax.iota(jnp.int32, 16)

    def fire_g(t, b):
        pltpu.async_copy(
            table_hbm.at[idx_all.at[pl.ds(t * BLK, BLK)]], rows.at[b], sg[b]
        )

    def drain_g(b):
        pltpu.make_async_copy(
            table_hbm.at[idx_all.at[pl.ds(0, BLK)]], rows.at[b], sg[b]
        ).wait()

    def transpose(b):
        def c_body(c0, carry):
            for co in range(4):
                c = c0 * 4 + co
                k = (c + lanes) & 63
                tkv = k >> 3
                kpi = ((k & 7) << 7) + lanes
                zer = jnp.zeros((16,), jnp.int32)
                for m in range(8):
                    v = plsc.load_gather(rows.at[b], [lanes + 16 * m, k])
                    plsc.store_scatter(planes.at[b], [tkv, zer, kpi + 16 * m], v)
            return carry

        lax.fori_loop(0, EMB_D // 4, c_body, 0)

    def fire_w(t, b):
        bl = bl0 + t
        j = bl // NTI
        ti = bl - j * NTI
        pltpu.async_copy(
            planes.at[b], out_hbm.at[pl.ds(j * 8, 8), pl.ds(ti, 1), :], so[b]
        )

    def drain_w(b):
        pltpu.make_async_copy(
            planes.at[b], out_hbm.at[pl.ds(0, 8), pl.ds(0, 1), :], so[b]
        ).wait()

    NPRO = NBUF + AHEAD
    for t in range(AHEAD):
        fire_g(t, t % NBUF)
    for t in range(NPRO):
        b = t % NBUF
        drain_g(b)
        fire_g(t + AHEAD, (t + AHEAD) % NBUF)
        if t >= NBUF:
            drain_w(b)
        transpose(b)
        fire_w(t, b)

    def loop_body(T, carry):
        t0 = NPRO + T * NBUF
        for i in range(NBUF):
            t = t0 + i
            b = (NPRO + i) % NBUF
            drain_g(b)
            fire_g(t + AHEAD, (NPRO + i + AHEAD) % NBUF)
            drain_w(b)
            transpose(b)
            fire_w(t, b)
        return carry

    lax.fori_loop(0, (BL_PER_W - NPRO - AHEAD) // NBUF, loop_body, 0)

    for t in range(BL_PER_W - AHEAD, BL_PER_W):
        b = t % NBUF
        drain_g(b)
        drain_w(b)
        transpose(b)
        fire_w(t, b)
    for t in range(BL_PER_W - NBUF, BL_PER_W):
        drain_w(t % NBUF)


def kernel(x, table):
    x_lin = x.astype(jnp.int32).T.reshape(-1)
    o5 = _emb_lookup(x_lin, table).reshape(SEQ, 8, NTI, 8, BLK)
    return o5.transpose((2, 4, 0, 1, 3)).reshape(BATCH, SEQ, EMB_D)


# speedup vs baseline: 2.3793x; 1.3392x over previous
import functools

import jax
import jax.numpy as jnp
from jax import lax
from jax.experimental import pallas as pl
from jax.experimental.pallas import tpu as pltpu
from jax.experimental.pallas import tpu_sc as plsc

NUM_EMB_ROWS = 1000001
EMB_D = 64
BATCH = 16384
SEQ = 50

NC = 2
NS = 16
NW = NC * NS

BLK = 128
NBLOCKS = SEQ * (BATCH // BLK)
BL_PER_W = NBLOCKS // NW
NTI = BATCH // BLK
NBUF = 4
AHEAD = 2


@functools.partial(
    pl.kernel,
    out_type=jax.ShapeDtypeStruct((SEQ * 8, NTI, 8 * BLK), jnp.float32),
    mesh=plsc.VectorSubcoreMesh(core_axis_name="c", subcore_axis_name="s"),
    scratch_types=[
        pltpu.VMEM((BL_PER_W * BLK,), jnp.int32),
        pltpu.VMEM((NBUF, BLK, EMB_D), jnp.float32),
        pltpu.VMEM((NBUF, 8, 1, 8 * BLK), jnp.float32),
        [pltpu.SemaphoreType.DMA] * NBUF,
        [pltpu.SemaphoreType.DMA] * NBUF,
    ],
    compiler_params=pltpu.CompilerParams(
        use_tc_tiling_on_sc=False, needs_layout_passes=False
    ),
)
def _emb_lookup(x_hbm, table_hbm, out_hbm, idx_all, rows, planes, sg, so):
    wid = lax.axis_index("s") * NC + lax.axis_index("c")
    bl0 = wid * BL_PER_W
    pltpu.sync_copy(x_hbm.at[pl.ds(bl0 * BLK, BL_PER_W * BLK)], idx_all)

    lanes = lax.iota(jnp.int32, 16)

    def fire_g(t, b):
        pltpu.async_copy(
            table_hbm.at[idx_all.at[pl.ds(t * BLK, BLK)]], rows.at[b], sg[b]
        )

    def drain_g(b):
        pltpu.make_async_copy(
            table_hbm.at[idx_all.at[pl.ds(0, BLK)]], rows.at[b], sg[b]
        ).wait()

    def transpose(b):
        def c_body(c0, carry):
            zer = jnp.zeros((16,), jnp.int32)
            cs = []
            for co in range(4):
                k = (c0 * 4 + co + lanes) & 63
                cs.append((k, k >> 3, ((k & 7) << 7) + lanes))
            for m in range(8):
                vs = [
                    plsc.load_gather(rows.at[b], [lanes + 16 * m, k])
                    for (k, _, _) in cs
                ]
                for (_, tkv, kpi), v in zip(cs, vs):
                    plsc.store_scatter(planes.at[b], [tkv, zer, kpi + 16 * m], v)
            return carry

        lax.fori_loop(0, EMB_D // 4, c_body, 0)

    def fire_w(t, b):
        bl = bl0 + t
        j = bl // NTI
        ti = bl - j * NTI
        pltpu.async_copy(
            planes.at[b], out_hbm.at[pl.ds(j * 8, 8), pl.ds(ti, 1), :], so[b]
        )

    def drain_w(b):
        pltpu.make_async_copy(
            planes.at[b], out_hbm.at[pl.ds(0, 8), pl.ds(0, 1), :], so[b]
        ).wait()

    NPRO = NBUF + AHEAD
    for t in range(AHEAD):
        fire_g(t, t % NBUF)
    for t in range(NPRO):
        b = t % NBUF
        drain_g(b)
        fire_g(t + AHEAD, (t + AHEAD) % NBUF)
        if t >= NBUF:
            drain_w(b)
        transpose(b)
        fire_w(t, b)

    def loop_body(T, carry):
        t0 = NPRO + T * NBUF
        for i in range(NBUF):
            t = t0 + i
            b = (NPRO + i) % NBUF
            drain_g(b)
            fire_g(t + AHEAD, (NPRO + i + AHEAD) % NBUF)
            drain_w(b)
            transpose(b)
            fire_w(t, b)
        return carry

    lax.fori_loop(0, (BL_PER_W - NPRO - AHEAD) // NBUF, loop_body, 0)

    for t in range(BL_PER_W - AHEAD, BL_PER_W):
        b = t % NBUF
        drain_g(b)
        drain_w(b)
        transpose(b)
        fire_w(t, b)
    for t in range(BL_PER_W - NBUF, BL_PER_W):
        drain_w(t % NBUF)


def kernel(x, table):
    x_lin = x.astype(jnp.int32).T.reshape(-1)
    o5 = _emb_lookup(x_lin, table).reshape(SEQ, 8, NTI, 8, BLK)
    return o5.transpose((2, 4, 0, 1, 3)).reshape(BATCH, SEQ, EMB_D)


# speedup vs baseline: 2.5949x; 1.0906x over previous
import functools

import jax
import jax.numpy as jnp
from jax import lax
from jax.experimental import pallas as pl
from jax.experimental.pallas import tpu as pltpu
from jax.experimental.pallas import tpu_sc as plsc

NUM_EMB_ROWS = 1000001
EMB_D = 64
BATCH = 16384
SEQ = 50

NC = 2
NS = 16
NW = NC * NS

BLK = 128
NBLOCKS = SEQ * (BATCH // BLK)
BL_PER_W = NBLOCKS // NW
NTI = BATCH // BLK
NBUF = 4
AHEAD = 2


@functools.partial(
    pl.kernel,
    out_type=jax.ShapeDtypeStruct((SEQ * 8, NTI, 8 * BLK), jnp.float32),
    mesh=plsc.VectorSubcoreMesh(core_axis_name="c", subcore_axis_name="s"),
    scratch_types=[
        pltpu.VMEM((BL_PER_W * BLK,), jnp.int32),
        pltpu.VMEM((NBUF, BLK, EMB_D), jnp.float32),
        pltpu.VMEM((NBUF, 8, 1, 8 * BLK), jnp.float32),
        [pltpu.SemaphoreType.DMA] * NBUF,
        [pltpu.SemaphoreType.DMA] * NBUF,
    ],
    compiler_params=pltpu.CompilerParams(
        use_tc_tiling_on_sc=False, needs_layout_passes=False
    ),
)
def _emb_lookup(x_hbm, table_hbm, out_hbm, idx_all, rows, planes, sg, so):
    wid = lax.axis_index("s") * NC + lax.axis_index("c")
    bl0 = wid * BL_PER_W
    pltpu.sync_copy(x_hbm.at[pl.ds(bl0 * BLK, BL_PER_W * BLK)], idx_all)

    lanes = lax.iota(jnp.int32, 16)

    def fire_g(t, b):
        pltpu.async_copy(
            table_hbm.at[idx_all.at[pl.ds(t * BLK, BLK)]], rows.at[b], sg[b]
        )

    def drain_g(b):
        pltpu.make_async_copy(
            table_hbm.at[idx_all.at[pl.ds(0, BLK)]], rows.at[b], sg[b]
        ).wait()

    def transpose(b):
        def c_body(c0, carry):
            zer = jnp.zeros((16,), jnp.int32)
            cs = []
            for co in range(4):
                k = (c0 * 4 + co + lanes) & 63
                cs.append((k, k >> 3, ((k & 7) << 7) + lanes))
            for m in range(8):
                vs = [
                    plsc.load_gather(rows.at[b], [lanes + 16 * m, k])
                    for (k, _, _) in cs
                ]
                for (_, tkv, kpi), v in zip(cs, vs):
                    plsc.store_scatter(planes.at[b], [tkv, zer, kpi + 16 * m], v)
            return carry

        lax.fori_loop(0, EMB_D // 4, c_body, 0)

    def fire_w(t, b):
        bl = bl0 + t
        j = bl // NTI
        ti = bl - j * NTI
        pltpu.async_copy(
            planes.at[b], out_hbm.at[pl.ds(j * 8, 8), pl.ds(ti, 1), :], so[b]
        )

    def drain_w(b):
        pltpu.make_async_copy(
            planes.at[b], out_hbm.at[pl.ds(0, 8), pl.ds(0, 1), :], so[b]
        ).wait()

    NPRO = NBUF + AHEAD
    for t in range(AHEAD):
        fire_g(t, t % NBUF)
    for t in range(NPRO):
        b = t % NBUF
        drain_g(b)
        fire_g(t + AHEAD, (t + AHEAD) % NBUF)
        if t >= NBUF:
            drain_w(b)
        transpose(b)
        fire_w(t, b)

    def loop_body(T, carry):
        t0 = NPRO + T * NBUF
        for i in range(NBUF):
            t = t0 + i
            b = (NPRO + i) % NBUF
            drain_g(b)
            fire_g(t + AHEAD, (NPRO + i + AHEAD) % NBUF)
            drain_w(b)
            transpose(b)
            fire_w(t, b)
        return carry

    lax.fori_loop(0, (BL_PER_W - NPRO - AHEAD) // NBUF, loop_body, 0)

    for t in range(BL_PER_W - AHEAD, BL_PER_W):
        b = t % NBUF
        drain_g(b)
        drain_w(b)
        transpose(b)
        fire_w(t, b)
    for t in range(BL_PER_W - NBUF, BL_PER_W):
        drain_w(t % NBUF)


def kernel(x, table):
    x_lin = (x.astype(jnp.int32) * 2).T.reshape(-1)
    t_lin = jnp.pad(table, ((0, 7), (0, EMB_D))).reshape(2 * (NUM_EMB_ROWS + 7), EMB_D)
    o5 = _emb_lookup(x_lin, t_lin).reshape(SEQ, 8, NTI, 8, BLK)
    return o5.transpose((2, 4, 0, 1, 3)).reshape(BATCH, SEQ, EMB_D)
